# baseline (speedup 1.0000x reference)
import functools
import math

import jax
import jax.numpy as jnp
from jax import lax
from jax.experimental import pallas as pl
from jax.experimental.pallas import tpu as pltpu
from jax.experimental.pallas import tpu_sc as plsc

B = 8
N = 2048
R = B * N
K = 16
NH = 8
D = 256
HD = D // NH
EPS = 1e-5
NW = 32


def _make_gather_max(C, with_post):
    rows_per_tile = R // NW
    P = 2048 // C
    G = (P * K) // 128
    n_rounds = rows_per_tile // P
    mesh = plsc.VectorSubcoreMesh(core_axis_name="c", subcore_axis_name="s")

    idx_rows = rows_per_tile * K // 128
    scratch = [
        pltpu.VMEM((idx_rows, 128), jnp.int32),
        pltpu.VMEM((P * K, C), jnp.float32),
        pltpu.VMEM((P, C), jnp.float32),
        pltpu.SemaphoreType.DMA,
    ]
    if with_post:
        scratch.append(pltpu.VMEM((P, C), jnp.float32))

    def body(table_hbm, idx_hbm, *rest):
        if with_post:
            v_hbm, out_hbm, idx_v, rows_v, out_v, sem, v_v = rest
        else:
            out_hbm, idx_v, rows_v, out_v, sem = rest[:5]
        wid = lax.axis_index("s") * 2 + lax.axis_index("c")
        base = wid * rows_per_tile
        pltpu.sync_copy(idx_hbm.at[pl.ds(wid * idx_rows, idx_rows)], idx_v)

        def round_body(r, _):
            pbase = base + r * P
            if with_post:
                pltpu.sync_copy(v_hbm.at[pl.ds(pbase, P)], v_v)
            copies = [
                pltpu.async_copy(
                    table_hbm.at[idx_v.at[r * G + g]],
                    rows_v.at[pl.ds(g * 128, 128)],
                    sem,
                )
                for g in range(G)
            ]
            for cp in copies:
                cp.wait()

            def point_body(p, _):
                for c in range(C // 16):
                    sl = pl.ds(c * 16, 16)
                    acc = rows_v[p * K, sl]
                    for k in range(1, K):
                        acc = jnp.maximum(acc, rows_v[p * K + k, sl])
                    if with_post:
                        acc = jnp.maximum(acc + v_v[p, sl], 0.0)
                    out_v[p, sl] = acc
                return 0

            lax.fori_loop(0, P, point_body, 0)
            pltpu.sync_copy(out_v, out_hbm.at[pl.ds(pbase, P)])
            return 0

        lax.fori_loop(0, n_rounds, round_body, 0)

    kern = pl.kernel(
        body,
        out_type=jax.ShapeDtypeStruct((R, C), jnp.float32),
        mesh=mesh,
        scratch_types=scratch,
        compiler_params=pltpu.CompilerParams(use_tc_tiling_on_sc=False, needs_layout_passes=False),
    )
    return kern


_make_gather_max = functools.lru_cache(None)(_make_gather_max)


def _gather_max_64(table, idx2d):
    return _make_gather_max(64, False)(table, idx2d)


def _gather_max_64_post(table, idx2d, v):
    return _make_gather_max(64, True)(table, idx2d, v)


def _gather_max_128(table, idx2d):
    return _make_gather_max(128, False)(table, idx2d)


def _score_body(xq_ref, xb_ref, s_ref):
    xq = xq_ref[0]
    xb = xb_ref[0]
    inner = lax.dot_general(xq, xb, (((1,), (1,)), ((), ())),
                            preferred_element_type=jnp.float32)
    q0, q1, q2 = xq[:, 0:1], xq[:, 1:2], xq[:, 2:3]
    nq = q0 * q0 + q1 * q1 + q2 * q2
    b0 = xb[:, 0:1].T
    b1 = xb[:, 1:2].T
    b2 = xb[:, 2:3].T
    nb = b0 * b0 + b1 * b1 + b2 * b2
    s_ref[...] = (2.0 * inner - nq) - nb


def _score_tc(x):
    TQ = 256
    nblk = N // TQ
    xq = x.reshape(B * nblk, TQ, 3)
    return pl.pallas_call(
        _score_body,
        grid=(B * nblk,),
        in_specs=[
            pl.BlockSpec((1, TQ, 3), lambda i: (i, 0, 0)),
            pl.BlockSpec((1, N, 3), lambda i: (i // nblk, 0, 0)),
        ],
        out_specs=pl.BlockSpec((TQ, N), lambda i: (i, 0)),
        out_shape=jax.ShapeDtypeStruct((R, N), jnp.float32),
    )(xq, x)


def _topk_body(s_hbm, out_hbm, rows_v, out_v, ck_v, cv_v, sem):
    rows_per_tile = R // NW
    GQ = 8
    n_groups = rows_per_tile // GQ
    wid = lax.axis_index("s") * 2 + lax.axis_index("c")
    base = wid * rows_per_tile
    iota = lax.iota(jnp.int32, 16)
    neg_inf = jnp.float32(-jnp.inf)

    def fold16(acc_k, acc_v, gk, gv):
        sk, sv = plsc.sort_key_val(gk, gv, descending=True)
        ra = lax.rev(acc_k, (0,))
        rv = lax.rev(acc_v, (0,))
        sel = sk > ra
        mk = jnp.where(sel, sk, ra)
        mv = jnp.where(sel, sv, rv)
        return plsc.sort_key_val(mk, mv, descending=True)

    def rebuild(acc_k, acc_v, cnt):
        for g in range(9):
            gk = ck_v[pl.ds(g * 16, 16)]
            gv = cv_v[pl.ds(g * 16, 16)]
            gk = jnp.where(iota + g * 16 < cnt, gk, neg_inf)
            acc_k, acc_v = fold16(acc_k, acc_v, gk, gv)
        return acc_k, acc_v

    def group_body(g, slot):
        @pl.when(g + 1 < n_groups)
        def _():
            pltpu.async_copy(
                s_hbm.at[pl.ds(base + (g + 1) * GQ, GQ)],
                rows_v.at[1 - slot], sem)

        def query_body(qi, _):
            q = g * GQ + qi
            r = base + q
            boff = (r // N) * N

            UNR = 8

            def chunk_body(c4, carry):
                acc_k, acc_v, thresh, cnt = carry
                ss, ms, pcs = [], [], []
                for u in range(UNR):
                    s = rows_v[slot, qi, pl.ds((c4 * UNR + u) * 16, 16)]
                    m = s > thresh
                    ss.append(s)
                    ms.append(m)
                    pcs.append(plsc.all_reduce_population_count(m)[0])
                for u in range(UNR):
                    idxv = iota + (boff + (c4 * UNR + u) * 16)
                    off = cnt if u == 0 else cnt + sum(pcs[:u])
                    plsc.store_compressed(ck_v.at[pl.ds(off, 16)], ss[u], mask=ms[u])
                    plsc.store_compressed(cv_v.at[pl.ds(off, 16)], idxv, mask=ms[u])
                ncnt = cnt + sum(pcs)

                def do_rebuild(args):
                    a_k, a_v, nc = args
                    a_k, a_v = rebuild(a_k, a_v, nc)
                    return a_k, a_v, a_k[15], jnp.int32(0)

                def no_rebuild(args):
                    a_k, a_v, nc = args
                    return a_k, a_v, thresh, nc

                return lax.cond(ncnt >= 16, do_rebuild, no_rebuild,
                                (acc_k, acc_v, ncnt))

            init = (jnp.full((16,), neg_inf, jnp.float32),
                    jnp.zeros((16,), jnp.int32),
                    neg_inf, jnp.int32(0))
            acc_k, acc_v, thresh, cnt = lax.fori_loop(0, N // 16 // UNR,
                                                      chunk_body, init)
            acc_k, acc_v = rebuild(acc_k, acc_v, cnt)
            out_v[(q // 8), pl.ds((q % 8) * 16, 16)] = acc_v
            return 0

        lax.fori_loop(0, GQ, query_body, 0)
        @pl.when(g + 1 < n_groups)
        def _():
            pltpu.make_async_copy(
                s_hbm.at[pl.ds(base + (g + 1) * GQ, GQ)],
                rows_v.at[1 - slot], sem).wait()
        return 1 - slot

    pltpu.async_copy(s_hbm.at[pl.ds(base, GQ)], rows_v.at[0], sem).wait()
    lax.fori_loop(0, n_groups, group_body, 0)
    pltpu.sync_copy(out_v, out_hbm.at[pl.ds(wid * (rows_per_tile * K // 128),
                                            rows_per_tile * K // 128)])


@functools.lru_cache(None)
def _make_topk():
    mesh = plsc.VectorSubcoreMesh(core_axis_name="c", subcore_axis_name="s")
    return pl.kernel(
        _topk_body,
        out_type=jax.ShapeDtypeStruct((R * K // 128, 128), jnp.int32),
        mesh=mesh,
        scratch_types=[
            pltpu.VMEM((2, 8, N), jnp.float32),
            pltpu.VMEM((R // NW * K // 128, 128), jnp.int32),
            pltpu.VMEM((144,), jnp.float32),
            pltpu.VMEM((144,), jnp.int32),
            pltpu.SemaphoreType.DMA,
        ],
        compiler_params=pltpu.CompilerParams(use_tc_tiling_on_sc=False, needs_layout_passes=False),
    )


def _topk_sc(scores):
    return _make_topk()(scores)


def _prep_body(x_ref, uw_ref, vw_ref, vb_ref, pw_ref, pb_ref, u_ref, v_ref, m_ref):
    xb = x_ref[...]
    x0, x1, x2 = xb[:, 0:1], xb[:, 1:2], xb[:, 2:3]

    def mm3(w_ref):
        w = w_ref[...]
        return x0 * w[0:1, :] + x1 * w[1:2, :] + x2 * w[2:3, :]

    u_ref[...] = mm3(uw_ref)
    v_ref[...] = mm3(vw_ref) + vb_ref[...][None, :]
    m_ref[...] = jnp.maximum(mm3(pw_ref) + pb_ref[...][None, :], 0.0)


def _prep_tc(x2, uw, vw, vb, pw, pb):
    T = 2048
    grid = (R // T,)
    return pl.pallas_call(
        _prep_body,
        grid=grid,
        in_specs=[
            pl.BlockSpec((T, 3), lambda i: (i, 0)),
            pl.BlockSpec((3, 64), lambda i: (0, 0)),
            pl.BlockSpec((3, 64), lambda i: (0, 0)),
            pl.BlockSpec((64,), lambda i: (0,)),
            pl.BlockSpec((3, 256), lambda i: (0, 0)),
            pl.BlockSpec((256,), lambda i: (0,)),
        ],
        out_specs=[
            pl.BlockSpec((T, 64), lambda i: (i, 0)),
            pl.BlockSpec((T, 64), lambda i: (i, 0)),
            pl.BlockSpec((T, 256), lambda i: (i, 0)),
        ],
        out_shape=[
            jax.ShapeDtypeStruct((R, 64), jnp.float32),
            jax.ShapeDtypeStruct((R, 64), jnp.float32),
            jax.ShapeDtypeStruct((R, 256), jnp.float32),
        ],
    )(x2, uw, vw, vb, pw, pb)


def _conv_body(m_ref, w_ref, b_ref, y_ref):
    y = lax.dot_general(m_ref[...], w_ref[...], (((1,), (0,)), ((), ())),
                        preferred_element_type=jnp.float32)
    y_ref[...] = jnp.maximum(y + b_ref[...][None, :], 0.0)


def _conv_tc(m, w, b):
    T = 2048
    cin, cout = w.shape
    return pl.pallas_call(
        _conv_body,
        grid=(R // T,),
        in_specs=[
            pl.BlockSpec((T, cin), lambda i: (i, 0)),
            pl.BlockSpec((cin, cout), lambda i: (0, 0)),
            pl.BlockSpec((cout,), lambda i: (0,)),
        ],
        out_specs=pl.BlockSpec((T, cout), lambda i: (i, 0)),
        out_shape=jax.ShapeDtypeStruct((R, cout), jnp.float32),
    )(m, w, b)


def _final_body(y1_ref, y2_ref, y3_ref, f1_ref, f2_ref, f3_ref, fb_ref, o_ref):
    dn = (((1,), (0,)), ((), ()))
    z = lax.dot_general(y1_ref[0], f1_ref[...], dn, preferred_element_type=jnp.float32)
    z = z + lax.dot_general(y2_ref[0], f2_ref[...], dn, preferred_element_type=jnp.float32)
    z = z + lax.dot_general(y3_ref[0], f3_ref[...], dn, preferred_element_type=jnp.float32)
    z = z + fb_ref[...][None, :]
    o_ref[...] = jnp.max(z, axis=0)[None, None, :]


def _final_tc(y1, y2, y3, f1, f2, f3, fb):
    return pl.pallas_call(
        _final_body,
        grid=(B,),
        in_specs=[
            pl.BlockSpec((1, N, 64), lambda b: (b, 0, 0)),
            pl.BlockSpec((1, N, 128), lambda b: (b, 0, 0)),
            pl.BlockSpec((1, N, 256), lambda b: (b, 0, 0)),
            pl.BlockSpec((64, 512), lambda b: (0, 0)),
            pl.BlockSpec((128, 512), lambda b: (0, 0)),
            pl.BlockSpec((256, 512), lambda b: (0, 0)),
            pl.BlockSpec((512,), lambda b: (0,)),
        ],
        out_specs=pl.BlockSpec((1, 1, 512), lambda b: (b, 0, 0)),
        out_shape=jax.ShapeDtypeStruct((B, 1, 512), jnp.float32),
    )(y1.reshape(B, N, 64), y2.reshape(B, N, 128), y3.reshape(B, N, 256),
      f1, f2, f3, fb).reshape(B, 512)


def _ln(x, g, b):
    m = jnp.mean(x, axis=-1, keepdims=True)
    xc = x - m
    v = jnp.mean(xc * xc, axis=-1, keepdims=True)
    return xc * lax.rsqrt(v + EPS) * g[None, :] + b[None, :]


def _softmax(x):
    m = jnp.max(x, axis=-1, keepdims=True)
    e = jnp.exp(x - m)
    return e / jnp.sum(e, axis=-1, keepdims=True)


def _xf_body(xm_ref, mem_ref,
             asa_ref, csa_ref, usa_ref, ksa_ref,
             aca_ref, cca_ref, uca_ref, kca_ref,
             w1_ref, b1_ref, w2_ref, b2_ref,
             ln_ref,
             wpc_ref, bpc_ref, cw_ref, cb_ref, lnf_ref,
             o_ref, h_scr):
    li = pl.program_id(0)
    bi = pl.program_id(1)
    dn = (((1,), (0,)), ((), ()))
    dnt = (((1,), (1,)), ((), ()))

    @pl.when(li == 0)
    def _():
        q = lax.dot_general(xm_ref[0], wpc_ref[...], dn,
                            preferred_element_type=jnp.float32)
        h_scr[bi] = jnp.maximum(q + bpc_ref[...][None, :], 0.0)

    h = h_scr[bi]

    def attn(hh_in, kv, a_ref, c_ref, u_ref, k_ref):
        o = jnp.zeros((16, D), jnp.float32)
        for i in range(NH):
            hq = lax.dot_general(hh_in, a_ref[0, i], dn,
                                 preferred_element_type=jnp.float32)
            hq = hq + c_ref[0, i][None, :]
            sc = lax.dot_general(hq, kv, dnt, preferred_element_type=jnp.float32)
            att = _softmax(sc)
            am = lax.dot_general(att, kv, dn, preferred_element_type=jnp.float32)
            o = o + lax.dot_general(am, u_ref[0, i], dn,
                                    preferred_element_type=jnp.float32)
        return o + k_ref[0, 0][None, :]

    ln = ln_ref[0]
    h = _ln(h + attn(h, h, asa_ref, csa_ref, usa_ref, ksa_ref), ln[0], ln[1])
    h = _ln(h + attn(h, mem_ref[0], aca_ref, cca_ref, uca_ref, kca_ref), ln[2], ln[3])
    ff = lax.dot_general(h, w1_ref[0], dn, preferred_element_type=jnp.float32)
    ff = jnp.maximum(ff + b1_ref[0, 0][None, :], 0.0)
    ff = lax.dot_general(ff, w2_ref[0], dn, preferred_element_type=jnp.float32)
    ff = ff + b2_ref[0, 0][None, :]
    h = _ln(h + ff, ln[4], ln[5])
    h_scr[bi] = h

    @pl.when(li == 3)
    def _():
        hf = _ln(h, lnf_ref[0], lnf_ref[1])
        o = lax.dot_general(hf, cw_ref[...], dn, preferred_element_type=jnp.float32)
        o_ref[...] = (o + cb_ref[...][None, :])[None]


def _xf_tc(xm, mem, asa, csa, usa, ksa, aca, cca, uca, kca,
           w1, b1, w2, b2, lnp, wpc, bpc, cw, cb, lnf):
    L = 4
    return pl.pallas_call(
        _xf_body,
        grid=(L, B),
        in_specs=[
            pl.BlockSpec((1, 16, 32), lambda l, b: (b, 0, 0)),
            pl.BlockSpec((1, N, D), lambda l, b: (b, 0, 0)),
            pl.BlockSpec((1, NH, D, D), lambda l, b: (l, 0, 0, 0)),
            pl.BlockSpec((1, NH, D), lambda l, b: (l, 0, 0)),
            pl.BlockSpec((1, NH, D, D), lambda l, b: (l, 0, 0, 0)),
            pl.BlockSpec((1, 1, D), lambda l, b: (l, 0, 0)),
            pl.BlockSpec((1, NH, D, D), lambda l, b: (l, 0, 0, 0)),
            pl.BlockSpec((1, NH, D), lambda l, b: (l, 0, 0)),
            pl.BlockSpec((1, NH, D, D), lambda l, b: (l, 0, 0, 0)),
            pl.BlockSpec((1, 1, D), lambda l, b: (l, 0, 0)),
            pl.BlockSpec((1, D, 512), lambda l, b: (l, 0, 0)),
            pl.BlockSpec((1, 1, 512), lambda l, b: (l, 0, 0)),
            pl.BlockSpec((1, 512, D), lambda l, b: (l, 0, 0)),
            pl.BlockSpec((1, 1, D), lambda l, b: (l, 0, 0)),
            pl.BlockSpec((1, 6, D), lambda l, b: (l, 0, 0)),
            pl.BlockSpec((32, D), lambda l, b: (0, 0)),
            pl.BlockSpec((D,), lambda l, b: (0,)),
            pl.BlockSpec((D, 32), lambda l, b: (0, 0)),
            pl.BlockSpec((32,), lambda l, b: (0,)),
            pl.BlockSpec((2, D), lambda l, b: (0, 0)),
        ],
        out_specs=pl.BlockSpec((1, 16, 32), lambda l, b: (b, 0, 0)),
        out_shape=jax.ShapeDtypeStruct((B, 16, 32), jnp.float32),
        scratch_shapes=[pltpu.VMEM((B, 16, D), jnp.float32)],
    )(xm, mem, asa, csa, usa, ksa, aca, cca, uca, kca,
      w1, b1, w2, b2, lnp, wpc, bpc, cw, cb, lnf)


def _fold_attn(L, pfx):
    scale = 1.0 / math.sqrt(HD)
    wq, wk, wv, wo = (L[pfx + '_Wq'], L[pfx + '_Wk'], L[pfx + '_Wv'], L[pfx + '_Wo'])
    bq, bv, bo = L[pfx + '_bq'], L[pfx + '_bv'], L[pfx + '_bo']
    wq_h = wq.reshape(NH, HD, D)
    wk_h = wk.reshape(NH, HD, D)
    wv_h = wv.reshape(NH, HD, D)
    wo_h = wo.T.reshape(NH, HD, D)
    a = jnp.einsum('hkd,hke->hde', wq_h, wk_h) * scale
    c = jnp.einsum('hk,hke->he', bq.reshape(NH, HD), wk_h) * scale
    u = jnp.einsum('hkd,hke->hde', wv_h, wo_h)
    kconst = bv @ wo.T + bo
    return a, c, u, kconst


def kernel(x, params):
    p = params

    eW, eb = p['edge_W'], p['edge_b']
    eg, ebeta = p['edge_g'], p['edge_beta']
    w1m = eW[:, :3] * eg[:, None]
    w2m = (eW[:, 3:] - eW[:, :3]) * eg[:, None]
    vb = eg * eb + ebeta
    uw = w1m.T
    vw = w2m.T
    pw = p['proj_input_W'].T
    pb = p['proj_input_b']

    convs = []
    for L in p['points_convs']:
        convs.append(((L['W'] * L['g'][:, None]).T, L['g'] * L['b'] + L['beta']))

    fW = p['final_W']
    f1, f2, f3 = fW[:, :64].T, fW[:, 64:192].T, fW[:, 192:448].T
    fb = p['final_b']

    asa, csa, usa, ksa = [], [], [], []
    aca, cca, uca, kca = [], [], [], []
    w1l, b1l, w2l, b2l, lnl = [], [], [], [], []
    for L in p['layers']:
        a, c, u, kc = _fold_attn(L, 'sa')
        asa.append(a); csa.append(c); usa.append(u); ksa.append(kc)
        a, c, u, kc = _fold_attn(L, 'ca')
        aca.append(a); cca.append(c); uca.append(u); kca.append(kc)
        w1l.append(L['ffn_W1'].T); b1l.append(L['ffn_b1'])
        w2l.append(L['ffn_W2'].T); b2l.append(L['ffn_b2'])
        lnl.append(jnp.stack([L['ln1_g'], L['ln1_b'], L['ln2_g'], L['ln2_b'],
                              L['ln3_g'], L['ln3_b']]))
    stk = jnp.stack
    asa, csa, usa, ksa = stk(asa), stk(csa), stk(usa), stk(ksa)[:, None]
    aca, cca, uca, kca = stk(aca), stk(cca), stk(uca), stk(kca)[:, None]
    w1l, b1l = stk(w1l), stk(b1l)[:, None]
    w2l, b2l, lnl = stk(w2l), stk(b2l)[:, None], stk(lnl)
    lnf = jnp.stack([p['lnf_g'], p['lnf_b']])

    scores = _score_tc(x)
    idx2d = _topk_sc(scores)

    x2 = x.reshape(R, 3)
    u, v, mem = _prep_tc(x2, uw, vw, vb, pw, pb)

    y1 = _gather_max_64_post(u, idx2d, v)
    m2 = _gather_max_64(y1, idx2d)
    y2 = _conv_tc(m2, convs[0][0], convs[0][1])
    m3 = _gather_max_128(y2, idx2d)
    y3 = _conv_tc(m3, convs[1][0], convs[1][1])

    x_max = _final_tc(y1, y2, y3, f1, f2, f3, fb)

    xm = x_max.reshape(B, 16, 32)
    memb = mem.reshape(B, N, D)
    out = _xf_tc(xm, memb, asa, csa, usa, ksa, aca, cca, uca, kca,
                 w1l, b1l, w2l, b2l, lnl,
                 p['proj_codes_W'].T, p['proj_codes_b'],
                 p['compress_W'].T, p['compress_b'], lnf)
    return out.reshape(B, 512)


# speedup vs baseline: 1.0688x; 1.0688x over previous
import functools
import math

import jax
import jax.numpy as jnp
from jax import lax
from jax.experimental import pallas as pl
from jax.experimental.pallas import tpu as pltpu
from jax.experimental.pallas import tpu_sc as plsc

B = 8
N = 2048
R = B * N
K = 16
NH = 8
D = 256
HD = D // NH
EPS = 1e-5
NW = 32


def _make_gather_max(C, with_post):
    rows_per_tile = R // NW
    P = 2048 // C
    G = (P * K) // 128
    n_rounds = rows_per_tile // P
    mesh = plsc.VectorSubcoreMesh(core_axis_name="c", subcore_axis_name="s")

    idx_rows = rows_per_tile * K // 128
    scratch = [
        pltpu.VMEM((idx_rows, 128), jnp.int32),
        pltpu.VMEM((P * K, C), jnp.float32),
        pltpu.VMEM((P, C), jnp.float32),
        pltpu.SemaphoreType.DMA,
    ]
    if with_post:
        scratch.append(pltpu.VMEM((P, C), jnp.float32))

    def body(table_hbm, idx_hbm, *rest):
        if with_post:
            v_hbm, out_hbm, idx_v, rows_v, out_v, sem, v_v = rest
        else:
            out_hbm, idx_v, rows_v, out_v, sem = rest[:5]
        wid = lax.axis_index("s") * 2 + lax.axis_index("c")
        base = wid * rows_per_tile
        pltpu.sync_copy(idx_hbm.at[pl.ds(wid * idx_rows, idx_rows)], idx_v)

        def round_body(r, _):
            pbase = base + r * P
            if with_post:
                pltpu.sync_copy(v_hbm.at[pl.ds(pbase, P)], v_v)
            copies = [
                pltpu.async_copy(
                    table_hbm.at[idx_v.at[r * G + g]],
                    rows_v.at[pl.ds(g * 128, 128)],
                    sem,
                )
                for g in range(G)
            ]
            for cp in copies:
                cp.wait()

            def point_body(p, _):
                for c in range(C // 16):
                    sl = pl.ds(c * 16, 16)
                    acc = rows_v[p * K, sl]
                    for k in range(1, K):
                        acc = jnp.maximum(acc, rows_v[p * K + k, sl])
                    if with_post:
                        acc = jnp.maximum(acc + v_v[p, sl], 0.0)
                    out_v[p, sl] = acc
                return 0

            lax.fori_loop(0, P, point_body, 0)
            pltpu.sync_copy(out_v, out_hbm.at[pl.ds(pbase, P)])
            return 0

        lax.fori_loop(0, n_rounds, round_body, 0)

    kern = pl.kernel(
        body,
        out_type=jax.ShapeDtypeStruct((R, C), jnp.float32),
        mesh=mesh,
        scratch_types=scratch,
        compiler_params=pltpu.CompilerParams(use_tc_tiling_on_sc=False, needs_layout_passes=False),
    )
    return kern


_make_gather_max = functools.lru_cache(None)(_make_gather_max)


def _gather_max_64(table, idx2d):
    return _make_gather_max(64, False)(table, idx2d)


def _gather_max_64_post(table, idx2d, v):
    return _make_gather_max(64, True)(table, idx2d, v)


def _gather_max_128(table, idx2d):
    return _make_gather_max(128, False)(table, idx2d)


def _score_body(xq_ref, xb_ref, s_ref):
    xq = xq_ref[0]
    xb = xb_ref[0]
    inner = lax.dot_general(xq, xb, (((1,), (1,)), ((), ())),
                            preferred_element_type=jnp.float32)
    q0, q1, q2 = xq[:, 0:1], xq[:, 1:2], xq[:, 2:3]
    nq = q0 * q0 + q1 * q1 + q2 * q2
    b0 = xb[:, 0:1].T
    b1 = xb[:, 1:2].T
    b2 = xb[:, 2:3].T
    nb = b0 * b0 + b1 * b1 + b2 * b2
    s_ref[...] = (2.0 * inner - nq) - nb


def _score_tc(x):
    TQ = 256
    nblk = N // TQ
    xq = x.reshape(B * nblk, TQ, 3)
    return pl.pallas_call(
        _score_body,
        grid=(B * nblk,),
        in_specs=[
            pl.BlockSpec((1, TQ, 3), lambda i: (i, 0, 0)),
            pl.BlockSpec((1, N, 3), lambda i: (i // nblk, 0, 0)),
        ],
        out_specs=pl.BlockSpec((TQ, N), lambda i: (i, 0)),
        out_shape=jax.ShapeDtypeStruct((R, N), jnp.float32),
    )(xq, x)


def _topk_body(s_hbm, out_hbm, rows_v, out_v, ck_v, cv_v, sem):
    rows_per_tile = R // NW
    GQ = 8
    n_groups = rows_per_tile // GQ
    wid = lax.axis_index("s") * 2 + lax.axis_index("c")
    base = wid * rows_per_tile
    iota = lax.iota(jnp.int32, 16)
    neg_inf = jnp.float32(-jnp.inf)

    def fold16(acc_k, acc_v, gk, gv):
        sk, sv = plsc.sort_key_val(gk, gv, descending=True)
        ra = lax.rev(acc_k, (0,))
        rv = lax.rev(acc_v, (0,))
        sel = sk > ra
        mk = jnp.where(sel, sk, ra)
        mv = jnp.where(sel, sv, rv)
        return plsc.sort_key_val(mk, mv, descending=True)

    def rebuild(acc_k, acc_v, cnt):
        for g in range(9):
            gk = ck_v[pl.ds(g * 16, 16)]
            gv = cv_v[pl.ds(g * 16, 16)]
            gk = jnp.where(iota + g * 16 < cnt, gk, neg_inf)
            acc_k, acc_v = fold16(acc_k, acc_v, gk, gv)
        return acc_k, acc_v

    def group_body(g, slot):
        @pl.when(g + 1 < n_groups)
        def _():
            pltpu.async_copy(
                s_hbm.at[pl.ds(base + (g + 1) * GQ, GQ)],
                rows_v.at[1 - slot], sem)

        def query_body(qi, _):
            q = g * GQ + qi
            r = base + q
            boff = (r // N) * N

            UNR = 8

            def chunk_body(c4, carry):
                acc_k, acc_v, thresh, cnt = carry
                ss, ms, pcs = [], [], []
                for u in range(UNR):
                    s = rows_v[slot, qi, pl.ds((c4 * UNR + u) * 16, 16)]
                    m = s > thresh
                    ss.append(s)
                    ms.append(m)
                    pcs.append(plsc.all_reduce_population_count(m)[0])
                for u in range(UNR):
                    idxv = iota + (boff + (c4 * UNR + u) * 16)
                    off = cnt if u == 0 else cnt + sum(pcs[:u])
                    plsc.store_compressed(ck_v.at[pl.ds(off, 16)], ss[u], mask=ms[u])
                    plsc.store_compressed(cv_v.at[pl.ds(off, 16)], idxv, mask=ms[u])
                ncnt = cnt + sum(pcs)

                def do_rebuild(args):
                    a_k, a_v, nc = args
                    a_k, a_v = rebuild(a_k, a_v, nc)
                    return a_k, a_v, a_k[15], jnp.int32(0)

                def no_rebuild(args):
                    a_k, a_v, nc = args
                    return a_k, a_v, thresh, nc

                return lax.cond(ncnt >= 16, do_rebuild, no_rebuild,
                                (acc_k, acc_v, ncnt))

            init = (jnp.full((16,), neg_inf, jnp.float32),
                    jnp.zeros((16,), jnp.int32),
                    neg_inf, jnp.int32(0))
            acc_k, acc_v, thresh, cnt = lax.fori_loop(0, N // 16 // UNR,
                                                      chunk_body, init)
            acc_k, acc_v = rebuild(acc_k, acc_v, cnt)
            out_v[(q // 8), pl.ds((q % 8) * 16, 16)] = acc_v
            return 0

        lax.fori_loop(0, GQ, query_body, 0)
        @pl.when(g + 1 < n_groups)
        def _():
            pltpu.make_async_copy(
                s_hbm.at[pl.ds(base + (g + 1) * GQ, GQ)],
                rows_v.at[1 - slot], sem).wait()
        return 1 - slot

    pltpu.async_copy(s_hbm.at[pl.ds(base, GQ)], rows_v.at[0], sem).wait()
    lax.fori_loop(0, n_groups, group_body, 0)
    pltpu.sync_copy(out_v, out_hbm.at[pl.ds(wid * (rows_per_tile * K // 128),
                                            rows_per_tile * K // 128)])


@functools.lru_cache(None)
def _make_topk():
    mesh = plsc.VectorSubcoreMesh(core_axis_name="c", subcore_axis_name="s")
    return pl.kernel(
        _topk_body,
        out_type=jax.ShapeDtypeStruct((R * K // 128, 128), jnp.int32),
        mesh=mesh,
        scratch_types=[
            pltpu.VMEM((2, 8, N), jnp.float32),
            pltpu.VMEM((R // NW * K // 128, 128), jnp.int32),
            pltpu.VMEM((144,), jnp.float32),
            pltpu.VMEM((144,), jnp.int32),
            pltpu.SemaphoreType.DMA,
        ],
        compiler_params=pltpu.CompilerParams(use_tc_tiling_on_sc=True, needs_layout_passes=False),
    )


def _topk_sc(scores):
    return _make_topk()(scores)


def _prep_body(x_ref, uw_ref, vw_ref, vb_ref, pw_ref, pb_ref, u_ref, v_ref, m_ref):
    xb = x_ref[...]
    x0, x1, x2 = xb[:, 0:1], xb[:, 1:2], xb[:, 2:3]

    def mm3(w_ref):
        w = w_ref[...]
        return x0 * w[0:1, :] + x1 * w[1:2, :] + x2 * w[2:3, :]

    u_ref[...] = mm3(uw_ref)
    v_ref[...] = mm3(vw_ref) + vb_ref[...][None, :]
    m_ref[...] = jnp.maximum(mm3(pw_ref) + pb_ref[...][None, :], 0.0)


def _prep_tc(x2, uw, vw, vb, pw, pb):
    T = 2048
    grid = (R // T,)
    return pl.pallas_call(
        _prep_body,
        grid=grid,
        in_specs=[
            pl.BlockSpec((T, 3), lambda i: (i, 0)),
            pl.BlockSpec((3, 64), lambda i: (0, 0)),
            pl.BlockSpec((3, 64), lambda i: (0, 0)),
            pl.BlockSpec((64,), lambda i: (0,)),
            pl.BlockSpec((3, 256), lambda i: (0, 0)),
            pl.BlockSpec((256,), lambda i: (0,)),
        ],
        out_specs=[
            pl.BlockSpec((T, 64), lambda i: (i, 0)),
            pl.BlockSpec((T, 64), lambda i: (i, 0)),
            pl.BlockSpec((T, 256), lambda i: (i, 0)),
        ],
        out_shape=[
            jax.ShapeDtypeStruct((R, 64), jnp.float32),
            jax.ShapeDtypeStruct((R, 64), jnp.float32),
            jax.ShapeDtypeStruct((R, 256), jnp.float32),
        ],
    )(x2, uw, vw, vb, pw, pb)


def _conv_body(m_ref, w_ref, b_ref, y_ref):
    y = lax.dot_general(m_ref[...], w_ref[...], (((1,), (0,)), ((), ())),
                        preferred_element_type=jnp.float32)
    y_ref[...] = jnp.maximum(y + b_ref[...][None, :], 0.0)


def _conv_tc(m, w, b):
    T = 2048
    cin, cout = w.shape
    return pl.pallas_call(
        _conv_body,
        grid=(R // T,),
        in_specs=[
            pl.BlockSpec((T, cin), lambda i: (i, 0)),
            pl.BlockSpec((cin, cout), lambda i: (0, 0)),
            pl.BlockSpec((cout,), lambda i: (0,)),
        ],
        out_specs=pl.BlockSpec((T, cout), lambda i: (i, 0)),
        out_shape=jax.ShapeDtypeStruct((R, cout), jnp.float32),
    )(m, w, b)


def _final_body(y1_ref, y2_ref, y3_ref, f1_ref, f2_ref, f3_ref, fb_ref, o_ref):
    dn = (((1,), (0,)), ((), ()))
    z = lax.dot_general(y1_ref[0], f1_ref[...], dn, preferred_element_type=jnp.float32)
    z = z + lax.dot_general(y2_ref[0], f2_ref[...], dn, preferred_element_type=jnp.float32)
    z = z + lax.dot_general(y3_ref[0], f3_ref[...], dn, preferred_element_type=jnp.float32)
    z = z + fb_ref[...][None, :]
    o_ref[...] = jnp.max(z, axis=0)[None, None, :]


def _final_tc(y1, y2, y3, f1, f2, f3, fb):
    return pl.pallas_call(
        _final_body,
        grid=(B,),
        in_specs=[
            pl.BlockSpec((1, N, 64), lambda b: (b, 0, 0)),
            pl.BlockSpec((1, N, 128), lambda b: (b, 0, 0)),
            pl.BlockSpec((1, N, 256), lambda b: (b, 0, 0)),
            pl.BlockSpec((64, 512), lambda b: (0, 0)),
            pl.BlockSpec((128, 512), lambda b: (0, 0)),
            pl.BlockSpec((256, 512), lambda b: (0, 0)),
            pl.BlockSpec((512,), lambda b: (0,)),
        ],
        out_specs=pl.BlockSpec((1, 1, 512), lambda b: (b, 0, 0)),
        out_shape=jax.ShapeDtypeStruct((B, 1, 512), jnp.float32),
    )(y1.reshape(B, N, 64), y2.reshape(B, N, 128), y3.reshape(B, N, 256),
      f1, f2, f3, fb).reshape(B, 512)


def _ln(x, g, b):
    m = jnp.mean(x, axis=-1, keepdims=True)
    xc = x - m
    v = jnp.mean(xc * xc, axis=-1, keepdims=True)
    return xc * lax.rsqrt(v + EPS) * g[None, :] + b[None, :]


def _softmax(x):
    m = jnp.max(x, axis=-1, keepdims=True)
    e = jnp.exp(x - m)
    return e / jnp.sum(e, axis=-1, keepdims=True)


def _xf_body(xm_ref, mem_ref,
             asa_ref, csa_ref, usa_ref, ksa_ref,
             aca_ref, cca_ref, uca_ref, kca_ref,
             w1_ref, b1_ref, w2_ref, b2_ref,
             ln_ref,
             wpc_ref, bpc_ref, cw_ref, cb_ref, lnf_ref,
             o_ref, h_scr):
    li = pl.program_id(0)
    bi = pl.program_id(1)
    dn = (((1,), (0,)), ((), ()))
    dnt = (((1,), (1,)), ((), ()))

    @pl.when(li == 0)
    def _():
        q = lax.dot_general(xm_ref[0], wpc_ref[...], dn,
                            preferred_element_type=jnp.float32)
        h_scr[bi] = jnp.maximum(q + bpc_ref[...][None, :], 0.0)

    h = h_scr[bi]

    def attn(hh_in, kv, a_ref, c_ref, u_ref, k_ref):
        o = jnp.zeros((16, D), jnp.float32)
        for i in range(NH):
            hq = lax.dot_general(hh_in, a_ref[0, i], dn,
                                 preferred_element_type=jnp.float32)
            hq = hq + c_ref[0, i][None, :]
            sc = lax.dot_general(hq, kv, dnt, preferred_element_type=jnp.float32)
            att = _softmax(sc)
            am = lax.dot_general(att, kv, dn, preferred_element_type=jnp.float32)
            o = o + lax.dot_general(am, u_ref[0, i], dn,
                                    preferred_element_type=jnp.float32)
        return o + k_ref[0, 0][None, :]

    ln = ln_ref[0]
    h = _ln(h + attn(h, h, asa_ref, csa_ref, usa_ref, ksa_ref), ln[0], ln[1])
    h = _ln(h + attn(h, mem_ref[0], aca_ref, cca_ref, uca_ref, kca_ref), ln[2], ln[3])
    ff = lax.dot_general(h, w1_ref[0], dn, preferred_element_type=jnp.float32)
    ff = jnp.maximum(ff + b1_ref[0, 0][None, :], 0.0)
    ff = lax.dot_general(ff, w2_ref[0], dn, preferred_element_type=jnp.float32)
    ff = ff + b2_ref[0, 0][None, :]
    h = _ln(h + ff, ln[4], ln[5])
    h_scr[bi] = h

    @pl.when(li == 3)
    def _():
        hf = _ln(h, lnf_ref[0], lnf_ref[1])
        o = lax.dot_general(hf, cw_ref[...], dn, preferred_element_type=jnp.float32)
        o_ref[...] = (o + cb_ref[...][None, :])[None]


def _xf_tc(xm, mem, asa, csa, usa, ksa, aca, cca, uca, kca,
           w1, b1, w2, b2, lnp, wpc, bpc, cw, cb, lnf):
    L = 4
    return pl.pallas_call(
        _xf_body,
        grid=(L, B),
        in_specs=[
            pl.BlockSpec((1, 16, 32), lambda l, b: (b, 0, 0)),
            pl.BlockSpec((1, N, D), lambda l, b: (b, 0, 0)),
            pl.BlockSpec((1, NH, D, D), lambda l, b: (l, 0, 0, 0)),
            pl.BlockSpec((1, NH, D), lambda l, b: (l, 0, 0)),
            pl.BlockSpec((1, NH, D, D), lambda l, b: (l, 0, 0, 0)),
            pl.BlockSpec((1, 1, D), lambda l, b: (l, 0, 0)),
            pl.BlockSpec((1, NH, D, D), lambda l, b: (l, 0, 0, 0)),
            pl.BlockSpec((1, NH, D), lambda l, b: (l, 0, 0)),
            pl.BlockSpec((1, NH, D, D), lambda l, b: (l, 0, 0, 0)),
            pl.BlockSpec((1, 1, D), lambda l, b: (l, 0, 0)),
            pl.BlockSpec((1, D, 512), lambda l, b: (l, 0, 0)),
            pl.BlockSpec((1, 1, 512), lambda l, b: (l, 0, 0)),
            pl.BlockSpec((1, 512, D), lambda l, b: (l, 0, 0)),
            pl.BlockSpec((1, 1, D), lambda l, b: (l, 0, 0)),
            pl.BlockSpec((1, 6, D), lambda l, b: (l, 0, 0)),
            pl.BlockSpec((32, D), lambda l, b: (0, 0)),
            pl.BlockSpec((D,), lambda l, b: (0,)),
            pl.BlockSpec((D, 32), lambda l, b: (0, 0)),
            pl.BlockSpec((32,), lambda l, b: (0,)),
            pl.BlockSpec((2, D), lambda l, b: (0, 0)),
        ],
        out_specs=pl.BlockSpec((1, 16, 32), lambda l, b: (b, 0, 0)),
        out_shape=jax.ShapeDtypeStruct((B, 16, 32), jnp.float32),
        scratch_shapes=[pltpu.VMEM((B, 16, D), jnp.float32)],
    )(xm, mem, asa, csa, usa, ksa, aca, cca, uca, kca,
      w1, b1, w2, b2, lnp, wpc, bpc, cw, cb, lnf)


def _fold_attn(L, pfx):
    scale = 1.0 / math.sqrt(HD)
    wq, wk, wv, wo = (L[pfx + '_Wq'], L[pfx + '_Wk'], L[pfx + '_Wv'], L[pfx + '_Wo'])
    bq, bv, bo = L[pfx + '_bq'], L[pfx + '_bv'], L[pfx + '_bo']
    wq_h = wq.reshape(NH, HD, D)
    wk_h = wk.reshape(NH, HD, D)
    wv_h = wv.reshape(NH, HD, D)
    wo_h = wo.T.reshape(NH, HD, D)
    a = jnp.einsum('hkd,hke->hde', wq_h, wk_h) * scale
    c = jnp.einsum('hk,hke->he', bq.reshape(NH, HD), wk_h) * scale
    u = jnp.einsum('hkd,hke->hde', wv_h, wo_h)
    kconst = bv @ wo.T + bo
    return a, c, u, kconst


def kernel(x, params):
    p = params

    eW, eb = p['edge_W'], p['edge_b']
    eg, ebeta = p['edge_g'], p['edge_beta']
    w1m = eW[:, :3] * eg[:, None]
    w2m = (eW[:, 3:] - eW[:, :3]) * eg[:, None]
    vb = eg * eb + ebeta
    uw = w1m.T
    vw = w2m.T
    pw = p['proj_input_W'].T
    pb = p['proj_input_b']

    convs = []
    for L in p['points_convs']:
        convs.append(((L['W'] * L['g'][:, None]).T, L['g'] * L['b'] + L['beta']))

    fW = p['final_W']
    f1, f2, f3 = fW[:, :64].T, fW[:, 64:192].T, fW[:, 192:448].T
    fb = p['final_b']

    asa, csa, usa, ksa = [], [], [], []
    aca, cca, uca, kca = [], [], [], []
    w1l, b1l, w2l, b2l, lnl = [], [], [], [], []
    for L in p['layers']:
        a, c, u, kc = _fold_attn(L, 'sa')
        asa.append(a); csa.append(c); usa.append(u); ksa.append(kc)
        a, c, u, kc = _fold_attn(L, 'ca')
        aca.append(a); cca.append(c); uca.append(u); kca.append(kc)
        w1l.append(L['ffn_W1'].T); b1l.append(L['ffn_b1'])
        w2l.append(L['ffn_W2'].T); b2l.append(L['ffn_b2'])
        lnl.append(jnp.stack([L['ln1_g'], L['ln1_b'], L['ln2_g'], L['ln2_b'],
                              L['ln3_g'], L['ln3_b']]))
    stk = jnp.stack
    asa, csa, usa, ksa = stk(asa), stk(csa), stk(usa), stk(ksa)[:, None]
    aca, cca, uca, kca = stk(aca), stk(cca), stk(uca), stk(kca)[:, None]
    w1l, b1l = stk(w1l), stk(b1l)[:, None]
    w2l, b2l, lnl = stk(w2l), stk(b2l)[:, None], stk(lnl)
    lnf = jnp.stack([p['lnf_g'], p['lnf_b']])

    scores = _score_tc(x)
    idx2d = _topk_sc(scores)

    x2 = x.reshape(R, 3)
    u, v, mem = _prep_tc(x2, uw, vw, vb, pw, pb)

    y1 = _gather_max_64_post(u, idx2d, v)
    m2 = _gather_max_64(y1, idx2d)
    y2 = _conv_tc(m2, convs[0][0], convs[0][1])
    m3 = _gather_max_128(y2, idx2d)
    y3 = _conv_tc(m3, convs[1][0], convs[1][1])

    x_max = _final_tc(y1, y2, y3, f1, f2, f3, fb)

    xm = x_max.reshape(B, 16, 32)
    memb = mem.reshape(B, N, D)
    out = _xf_tc(xm, memb, asa, csa, usa, ksa, aca, cca, uca, kca,
                 w1l, b1l, w2l, b2l, lnl,
                 p['proj_codes_W'].T, p['proj_codes_b'],
                 p['compress_W'].T, p['compress_b'], lnf)
    return out.reshape(B, 512)


# speedup vs baseline: 1.2571x; 1.1762x over previous
import functools
import math

import jax
import jax.numpy as jnp
from jax import lax
from jax.experimental import pallas as pl
from jax.experimental.pallas import tpu as pltpu
from jax.experimental.pallas import tpu_sc as plsc

B = 8
N = 2048
R = B * N
K = 16
NH = 8
D = 256
HD = D // NH
EPS = 1e-5
NW = 32


def _make_gather_max(C, with_post):
    rows_per_tile = R // NW
    P = 2048 // C
    G = (P * K) // 128
    n_rounds = rows_per_tile // P
    mesh = plsc.VectorSubcoreMesh(core_axis_name="c", subcore_axis_name="s")

    idx_rows = rows_per_tile * K // 128
    scratch = [
        pltpu.VMEM((idx_rows, 128), jnp.int32),
        pltpu.VMEM((P * K, C), jnp.float32),
        pltpu.VMEM((P, C), jnp.float32),
        pltpu.SemaphoreType.DMA,
    ]
    if with_post:
        scratch.append(pltpu.VMEM((P, C), jnp.float32))

    def body(table_hbm, idx_hbm, *rest):
        if with_post:
            v_hbm, out_hbm, idx_v, rows_v, out_v, sem, v_v = rest
        else:
            out_hbm, idx_v, rows_v, out_v, sem = rest[:5]
        wid = lax.axis_index("s") * 2 + lax.axis_index("c")
        base = wid * rows_per_tile
        pltpu.sync_copy(idx_hbm.at[pl.ds(wid * idx_rows, idx_rows)], idx_v)

        def round_body(r, _):
            pbase = base + r * P
            if with_post:
                pltpu.sync_copy(v_hbm.at[pl.ds(pbase, P)], v_v)
            copies = [
                pltpu.async_copy(
                    table_hbm.at[idx_v.at[r * G + g]],
                    rows_v.at[pl.ds(g * 128, 128)],
                    sem,
                )
                for g in range(G)
            ]
            for cp in copies:
                cp.wait()

            def point_body(p, _):
                for c in range(C // 16):
                    sl = pl.ds(c * 16, 16)
                    acc = rows_v[p * K, sl]
                    for k in range(1, K):
                        acc = jnp.maximum(acc, rows_v[p * K + k, sl])
                    if with_post:
                        acc = jnp.maximum(acc + v_v[p, sl], 0.0)
                    out_v[p, sl] = acc
                return 0

            lax.fori_loop(0, P, point_body, 0)
            pltpu.sync_copy(out_v, out_hbm.at[pl.ds(pbase, P)])
            return 0

        lax.fori_loop(0, n_rounds, round_body, 0)

    kern = pl.kernel(
        body,
        out_type=jax.ShapeDtypeStruct((R, C), jnp.float32),
        mesh=mesh,
        scratch_types=scratch,
        compiler_params=pltpu.CompilerParams(use_tc_tiling_on_sc=False, needs_layout_passes=False),
    )
    return kern


_make_gather_max = functools.lru_cache(None)(_make_gather_max)


def _gather_max_64(table, idx2d):
    return _make_gather_max(64, False)(table, idx2d)


def _gather_max_64_post(table, idx2d, v):
    return _make_gather_max(64, True)(table, idx2d, v)


def _gather_max_128(table, idx2d):
    return _make_gather_max(128, False)(table, idx2d)


def _score_body(xq_ref, xb_ref, s_ref):
    xq = xq_ref[0]
    xb = xb_ref[0]
    inner = lax.dot_general(xq, xb, (((1,), (1,)), ((), ())),
                            preferred_element_type=jnp.float32)
    q0, q1, q2 = xq[:, 0:1], xq[:, 1:2], xq[:, 2:3]
    nq = q0 * q0 + q1 * q1 + q2 * q2
    b0 = xb[:, 0:1].T
    b1 = xb[:, 1:2].T
    b2 = xb[:, 2:3].T
    nb = b0 * b0 + b1 * b1 + b2 * b2
    s_ref[...] = (2.0 * inner - nq) - nb


def _score_tc(x):
    TQ = 256
    nblk = N // TQ
    xq = x.reshape(B * nblk, TQ, 3)
    return pl.pallas_call(
        _score_body,
        grid=(B * nblk,),
        in_specs=[
            pl.BlockSpec((1, TQ, 3), lambda i: (i, 0, 0)),
            pl.BlockSpec((1, N, 3), lambda i: (i // nblk, 0, 0)),
        ],
        out_specs=pl.BlockSpec((TQ, N), lambda i: (i, 0)),
        out_shape=jax.ShapeDtypeStruct((R, N), jnp.float32),
    )(xq, x)


def _topk_body(s_hbm, out_hbm, rows_v, out_v, ck_v, cv_v, sem):
    rows_per_tile = R // NW
    GQ = 8
    n_groups = rows_per_tile // GQ
    wid = lax.axis_index("s") * 2 + lax.axis_index("c")
    base = wid * rows_per_tile
    iota = lax.iota(jnp.int32, 16)
    neg_inf = jnp.float32(-jnp.inf)

    def fold16(acc_k, acc_v, gk, gv):
        sk, sv = plsc.sort_key_val(gk, gv, descending=True)
        ra = lax.rev(acc_k, (0,))
        rv = lax.rev(acc_v, (0,))
        sel = sk > ra
        mk = jnp.where(sel, sk, ra)
        mv = jnp.where(sel, sv, rv)
        return plsc.sort_key_val(mk, mv, descending=True)

    def rebuild(acc_k, acc_v, cnt):
        for g in range(9):
            gk = ck_v[pl.ds(g * 16, 16)]
            gv = cv_v[pl.ds(g * 16, 16)]
            gk = jnp.where(iota + g * 16 < cnt, gk, neg_inf)
            acc_k, acc_v = fold16(acc_k, acc_v, gk, gv)
        return acc_k, acc_v

    def group_body(g, slot):
        @pl.when(g + 1 < n_groups)
        def _():
            pltpu.async_copy(
                s_hbm.at[pl.ds(base + (g + 1) * GQ, GQ)],
                rows_v.at[1 - slot], sem)

        def query_body(qi, _):
            q = g * GQ + qi
            r = base + q
            boff = (r // N) * N

            UNR = 8

            def chunk_body(c4, carry):
                acc_k, acc_v, thresh, cnt = carry
                ss, ms, pcs = [], [], []
                for u in range(UNR):
                    s = rows_v[slot, qi, pl.ds((c4 * UNR + u) * 16, 16)]
                    m = s > thresh
                    ss.append(s)
                    ms.append(m)
                    pcs.append(plsc.all_reduce_population_count(m)[0])
                for u in range(UNR):
                    idxv = iota + (boff + (c4 * UNR + u) * 16)
                    off = cnt if u == 0 else cnt + sum(pcs[:u])
                    plsc.store_compressed(ck_v.at[pl.ds(off, 16)], ss[u], mask=ms[u])
                    plsc.store_compressed(cv_v.at[pl.ds(off, 16)], idxv, mask=ms[u])
                ncnt = cnt + sum(pcs)

                def do_rebuild(args):
                    a_k, a_v, nc = args
                    a_k, a_v = rebuild(a_k, a_v, nc)
                    return a_k, a_v, a_k[15], jnp.int32(0)

                def no_rebuild(args):
                    a_k, a_v, nc = args
                    return a_k, a_v, thresh, nc

                return lax.cond(ncnt >= 16, do_rebuild, no_rebuild,
                                (acc_k, acc_v, ncnt))

            init = (jnp.full((16,), neg_inf, jnp.float32),
                    jnp.zeros((16,), jnp.int32),
                    neg_inf, jnp.int32(0))
            acc_k, acc_v, thresh, cnt = lax.fori_loop(0, N // 16 // UNR,
                                                      chunk_body, init)
            acc_k, acc_v = rebuild(acc_k, acc_v, cnt)
            out_v[(q // 8), pl.ds((q % 8) * 16, 16)] = acc_v
            return 0

        lax.fori_loop(0, GQ, query_body, 0)
        @pl.when(g + 1 < n_groups)
        def _():
            pltpu.make_async_copy(
                s_hbm.at[pl.ds(base + (g + 1) * GQ, GQ)],
                rows_v.at[1 - slot], sem).wait()
        return 1 - slot

    pltpu.async_copy(s_hbm.at[pl.ds(base, GQ)], rows_v.at[0], sem).wait()
    lax.fori_loop(0, n_groups, group_body, 0)
    pltpu.sync_copy(out_v, out_hbm.at[pl.ds(wid * (rows_per_tile * K // 128),
                                            rows_per_tile * K // 128)])


@functools.lru_cache(None)
def _make_topk():
    mesh = plsc.VectorSubcoreMesh(core_axis_name="c", subcore_axis_name="s")
    return pl.kernel(
        _topk_body,
        out_type=jax.ShapeDtypeStruct((R * K // 128, 128), jnp.int32),
        mesh=mesh,
        scratch_types=[
            pltpu.VMEM((2, 8, N), jnp.float32),
            pltpu.VMEM((R // NW * K // 128, 128), jnp.int32),
            pltpu.VMEM((144,), jnp.float32),
            pltpu.VMEM((144,), jnp.int32),
            pltpu.SemaphoreType.DMA,
        ],
        compiler_params=pltpu.CompilerParams(use_tc_tiling_on_sc=True, needs_layout_passes=False),
    )


def _topk_sc(scores):
    return _make_topk()(scores)


def _prep_body(x_ref, uw_ref, vw_ref, vb_ref, pw_ref, pb_ref, u_ref, v_ref, m_ref):
    xb = x_ref[...]
    x0, x1, x2 = xb[:, 0:1], xb[:, 1:2], xb[:, 2:3]

    def mm3(w_ref):
        w = w_ref[...]
        return x0 * w[0:1, :] + x1 * w[1:2, :] + x2 * w[2:3, :]

    u_ref[...] = mm3(uw_ref)
    v_ref[...] = mm3(vw_ref) + vb_ref[...][None, :]
    m_ref[...] = jnp.maximum(mm3(pw_ref) + pb_ref[...][None, :], 0.0)


def _prep_tc(x2, uw, vw, vb, pw, pb):
    T = 2048
    grid = (R // T,)
    return pl.pallas_call(
        _prep_body,
        grid=grid,
        in_specs=[
            pl.BlockSpec((T, 3), lambda i: (i, 0)),
            pl.BlockSpec((3, 64), lambda i: (0, 0)),
            pl.BlockSpec((3, 64), lambda i: (0, 0)),
            pl.BlockSpec((64,), lambda i: (0,)),
            pl.BlockSpec((3, 256), lambda i: (0, 0)),
            pl.BlockSpec((256,), lambda i: (0,)),
        ],
        out_specs=[
            pl.BlockSpec((T, 64), lambda i: (i, 0)),
            pl.BlockSpec((T, 64), lambda i: (i, 0)),
            pl.BlockSpec((T, 256), lambda i: (i, 0)),
        ],
        out_shape=[
            jax.ShapeDtypeStruct((R, 64), jnp.float32),
            jax.ShapeDtypeStruct((R, 64), jnp.float32),
            jax.ShapeDtypeStruct((R, 256), jnp.float32),
        ],
    )(x2, uw, vw, vb, pw, pb)


def _conv_body(m_ref, w_ref, b_ref, y_ref):
    y = lax.dot_general(m_ref[...], w_ref[...], (((1,), (0,)), ((), ())),
                        preferred_element_type=jnp.float32)
    y_ref[...] = jnp.maximum(y + b_ref[...][None, :], 0.0)


def _conv_tc(m, w, b):
    T = 2048
    cin, cout = w.shape
    return pl.pallas_call(
        _conv_body,
        grid=(R // T,),
        in_specs=[
            pl.BlockSpec((T, cin), lambda i: (i, 0)),
            pl.BlockSpec((cin, cout), lambda i: (0, 0)),
            pl.BlockSpec((cout,), lambda i: (0,)),
        ],
        out_specs=pl.BlockSpec((T, cout), lambda i: (i, 0)),
        out_shape=jax.ShapeDtypeStruct((R, cout), jnp.float32),
    )(m, w, b)


def _final_body(y1_ref, y2_ref, y3_ref, f1_ref, f2_ref, f3_ref, fb_ref, o_ref):
    dn = (((1,), (0,)), ((), ()))
    z = lax.dot_general(y1_ref[0], f1_ref[...], dn, preferred_element_type=jnp.float32)
    z = z + lax.dot_general(y2_ref[0], f2_ref[...], dn, preferred_element_type=jnp.float32)
    z = z + lax.dot_general(y3_ref[0], f3_ref[...], dn, preferred_element_type=jnp.float32)
    z = z + fb_ref[...][None, :]
    o_ref[...] = jnp.max(z, axis=0)[None, None, :]


def _final_tc(y1, y2, y3, f1, f2, f3, fb):
    return pl.pallas_call(
        _final_body,
        grid=(B,),
        in_specs=[
            pl.BlockSpec((1, N, 64), lambda b: (b, 0, 0)),
            pl.BlockSpec((1, N, 128), lambda b: (b, 0, 0)),
            pl.BlockSpec((1, N, 256), lambda b: (b, 0, 0)),
            pl.BlockSpec((64, 512), lambda b: (0, 0)),
            pl.BlockSpec((128, 512), lambda b: (0, 0)),
            pl.BlockSpec((256, 512), lambda b: (0, 0)),
            pl.BlockSpec((512,), lambda b: (0,)),
        ],
        out_specs=pl.BlockSpec((1, 1, 512), lambda b: (b, 0, 0)),
        out_shape=jax.ShapeDtypeStruct((B, 1, 512), jnp.float32),
    )(y1.reshape(B, N, 64), y2.reshape(B, N, 128), y3.reshape(B, N, 256),
      f1, f2, f3, fb).reshape(B, 512)


def _ln(x, g, b):
    m = jnp.mean(x, axis=-1, keepdims=True)
    xc = x - m
    v = jnp.mean(xc * xc, axis=-1, keepdims=True)
    return xc * lax.rsqrt(v + EPS) * g[None, :] + b[None, :]


def _softmax(x):
    m = jnp.max(x, axis=-1, keepdims=True)
    e = jnp.exp(x - m)
    return e / jnp.sum(e, axis=-1, keepdims=True)


def _xf_body(xm_ref, mem_ref,
             asa_ref, csa_ref, usa_ref, ksa_ref,
             aca_ref, cca_ref, uca_ref, kca_ref,
             w1_ref, b1_ref, w2_ref, b2_ref,
             ln_ref,
             wpc_ref, bpc_ref, cw_ref, cb_ref, lnf_ref,
             o_ref, h_scr):
    li = pl.program_id(0)
    bi = pl.program_id(1)
    dn = (((1,), (0,)), ((), ()))
    dnt = (((1,), (1,)), ((), ()))

    @pl.when(li == 0)
    def _():
        q = lax.dot_general(xm_ref[0], wpc_ref[...], dn,
                            preferred_element_type=jnp.float32)
        h_scr[bi] = jnp.maximum(q + bpc_ref[...][None, :], 0.0)

    h = h_scr[bi]

    def attn(hh_in, kv, a_ref, c_ref, u_ref, k_ref):
        hq_w = lax.dot_general(hh_in, a_ref[0], dn,
                               preferred_element_type=jnp.float32)
        hq_w = hq_w + c_ref[0, 0][None, :]
        hq = hq_w.reshape(16, NH, D).transpose(1, 0, 2).reshape(NH * 16, D)
        sc = lax.dot_general(hq, kv, dnt, preferred_element_type=jnp.float32)
        att = _softmax(sc)
        am = lax.dot_general(att, kv, dn, preferred_element_type=jnp.float32)
        am_c = am.reshape(NH, 16, D).transpose(1, 0, 2).reshape(16, NH * D)
        o = lax.dot_general(am_c, u_ref[0], dn, preferred_element_type=jnp.float32)
        return o + k_ref[0, 0][None, :]

    ln = ln_ref[0]
    h = _ln(h + attn(h, h, asa_ref, csa_ref, usa_ref, ksa_ref), ln[0], ln[1])
    h = _ln(h + attn(h, mem_ref[0], aca_ref, cca_ref, uca_ref, kca_ref), ln[2], ln[3])
    ff = lax.dot_general(h, w1_ref[0], dn, preferred_element_type=jnp.float32)
    ff = jnp.maximum(ff + b1_ref[0, 0][None, :], 0.0)
    ff = lax.dot_general(ff, w2_ref[0], dn, preferred_element_type=jnp.float32)
    ff = ff + b2_ref[0, 0][None, :]
    h = _ln(h + ff, ln[4], ln[5])
    h_scr[bi] = h

    @pl.when(li == 3)
    def _():
        hf = _ln(h, lnf_ref[0], lnf_ref[1])
        o = lax.dot_general(hf, cw_ref[...], dn, preferred_element_type=jnp.float32)
        o_ref[...] = (o + cb_ref[...][None, :])[None]


def _xf_tc(xm, mem, asa, csa, usa, ksa, aca, cca, uca, kca,
           w1, b1, w2, b2, lnp, wpc, bpc, cw, cb, lnf):
    L = 4
    return pl.pallas_call(
        _xf_body,
        grid=(L, B),
        in_specs=[
            pl.BlockSpec((1, 16, 32), lambda l, b: (b, 0, 0)),
            pl.BlockSpec((1, N, D), lambda l, b: (b, 0, 0)),
            pl.BlockSpec((1, D, NH * D), lambda l, b: (l, 0, 0)),
            pl.BlockSpec((1, 1, NH * D), lambda l, b: (l, 0, 0)),
            pl.BlockSpec((1, NH * D, D), lambda l, b: (l, 0, 0)),
            pl.BlockSpec((1, 1, D), lambda l, b: (l, 0, 0)),
            pl.BlockSpec((1, D, NH * D), lambda l, b: (l, 0, 0)),
            pl.BlockSpec((1, 1, NH * D), lambda l, b: (l, 0, 0)),
            pl.BlockSpec((1, NH * D, D), lambda l, b: (l, 0, 0)),
            pl.BlockSpec((1, 1, D), lambda l, b: (l, 0, 0)),
            pl.BlockSpec((1, D, 512), lambda l, b: (l, 0, 0)),
            pl.BlockSpec((1, 1, 512), lambda l, b: (l, 0, 0)),
            pl.BlockSpec((1, 512, D), lambda l, b: (l, 0, 0)),
            pl.BlockSpec((1, 1, D), lambda l, b: (l, 0, 0)),
            pl.BlockSpec((1, 6, D), lambda l, b: (l, 0, 0)),
            pl.BlockSpec((32, D), lambda l, b: (0, 0)),
            pl.BlockSpec((D,), lambda l, b: (0,)),
            pl.BlockSpec((D, 32), lambda l, b: (0, 0)),
            pl.BlockSpec((32,), lambda l, b: (0,)),
            pl.BlockSpec((2, D), lambda l, b: (0, 0)),
        ],
        out_specs=pl.BlockSpec((1, 16, 32), lambda l, b: (b, 0, 0)),
        out_shape=jax.ShapeDtypeStruct((B, 16, 32), jnp.float32),
        scratch_shapes=[pltpu.VMEM((B, 16, D), jnp.float32)],
    )(xm, mem, asa, csa, usa, ksa, aca, cca, uca, kca,
      w1, b1, w2, b2, lnp, wpc, bpc, cw, cb, lnf)


def _fold_attn(L, pfx):
    scale = 1.0 / math.sqrt(HD)
    wq, wk, wv, wo = (L[pfx + '_Wq'], L[pfx + '_Wk'], L[pfx + '_Wv'], L[pfx + '_Wo'])
    bq, bv, bo = L[pfx + '_bq'], L[pfx + '_bv'], L[pfx + '_bo']
    wq_h = wq.reshape(NH, HD, D)
    wk_h = wk.reshape(NH, HD, D)
    wv_h = wv.reshape(NH, HD, D)
    wo_h = wo.T.reshape(NH, HD, D)
    a = jnp.einsum('hkd,hke->hde', wq_h, wk_h) * scale
    c = jnp.einsum('hk,hke->he', bq.reshape(NH, HD), wk_h) * scale
    u = jnp.einsum('hkd,hke->hde', wv_h, wo_h)
    kconst = bv @ wo.T + bo
    return a, c, u, kconst


def kernel(x, params):
    p = params

    eW, eb = p['edge_W'], p['edge_b']
    eg, ebeta = p['edge_g'], p['edge_beta']
    w1m = eW[:, :3] * eg[:, None]
    w2m = (eW[:, 3:] - eW[:, :3]) * eg[:, None]
    vb = eg * eb + ebeta
    uw = w1m.T
    vw = w2m.T
    pw = p['proj_input_W'].T
    pb = p['proj_input_b']

    convs = []
    for L in p['points_convs']:
        convs.append(((L['W'] * L['g'][:, None]).T, L['g'] * L['b'] + L['beta']))

    fW = p['final_W']
    f1, f2, f3 = fW[:, :64].T, fW[:, 64:192].T, fW[:, 192:448].T
    fb = p['final_b']

    asa, csa, usa, ksa = [], [], [], []
    aca, cca, uca, kca = [], [], [], []
    w1l, b1l, w2l, b2l, lnl = [], [], [], [], []
    for L in p['layers']:
        a, c, u, kc = _fold_attn(L, 'sa')
        asa.append(a.transpose(1, 0, 2).reshape(D, NH * D))
        csa.append(c.reshape(1, NH * D)); usa.append(u.reshape(NH * D, D))
        ksa.append(kc)
        a, c, u, kc = _fold_attn(L, 'ca')
        aca.append(a.transpose(1, 0, 2).reshape(D, NH * D))
        cca.append(c.reshape(1, NH * D)); uca.append(u.reshape(NH * D, D))
        kca.append(kc)
        w1l.append(L['ffn_W1'].T); b1l.append(L['ffn_b1'])
        w2l.append(L['ffn_W2'].T); b2l.append(L['ffn_b2'])
        lnl.append(jnp.stack([L['ln1_g'], L['ln1_b'], L['ln2_g'], L['ln2_b'],
                              L['ln3_g'], L['ln3_b']]))
    stk = jnp.stack
    asa, csa, usa, ksa = stk(asa), stk(csa), stk(usa), stk(ksa)[:, None]
    aca, cca, uca, kca = stk(aca), stk(cca), stk(uca), stk(kca)[:, None]
    w1l, b1l = stk(w1l), stk(b1l)[:, None]
    w2l, b2l, lnl = stk(w2l), stk(b2l)[:, None], stk(lnl)
    lnf = jnp.stack([p['lnf_g'], p['lnf_b']])

    scores = _score_tc(x)
    idx2d = _topk_sc(scores)

    x2 = x.reshape(R, 3)
    u, v, mem = _prep_tc(x2, uw, vw, vb, pw, pb)

    y1 = _gather_max_64_post(u, idx2d, v)
    m2 = _gather_max_64(y1, idx2d)
    y2 = _conv_tc(m2, convs[0][0], convs[0][1])
    m3 = _gather_max_128(y2, idx2d)
    y3 = _conv_tc(m3, convs[1][0], convs[1][1])

    x_max = _final_tc(y1, y2, y3, f1, f2, f3, fb)

    xm = x_max.reshape(B, 16, 32)
    memb = mem.reshape(B, N, D)
    out = _xf_tc(xm, memb, asa, csa, usa, ksa, aca, cca, uca, kca,
                 w1l, b1l, w2l, b2l, lnl,
                 p['proj_codes_W'].T, p['proj_codes_b'],
                 p['compress_W'].T, p['compress_b'], lnf)
    return out.reshape(B, 512)
